# Initial kernel scaffold; baseline (speedup 1.0000x reference)
import jax
import jax.numpy as jnp
from jax.experimental import pallas as pl


def kernel(node_features, edge_features, edge_index, batch, W_in, b_in, W1, b1, g1, be1, W2, b2, g2, be2, W_out, b_out):
    raise NotImplementedError("write your pallas kernel here")


# speedup vs baseline: 5.8855x; 5.8855x over previous
import functools

import jax
import jax.numpy as jnp
from jax import lax
from jax.experimental import pallas as pl
from jax.experimental.pallas import tpu as pltpu
from jax.experimental.pallas import tpu_sc as plsc

N = 10000
E = 320000
D = 128
G = 64

NC = 2
NS = 16
NW = NC * NS

CHUNK = 128
NCH = 79
PER_TILE = NCH * CHUNK
E_PAD = NW * PER_TILE
N_PAD = 10240
RPT = N_PAD // NS

@functools.cache
def _make_sc_scatter_add():
    mesh = plsc.VectorSubcoreMesh(
        core_axis_name="c", subcore_axis_name="s", num_cores=NC, num_subcores=NS
    )

    @functools.partial(
        pl.kernel,
        out_type=jax.ShapeDtypeStruct((NC, N_PAD, D), jnp.float32),
        mesh=mesh,
        scratch_types=[
            pltpu.VMEM((CHUNK,), jnp.int32),
            pltpu.VMEM((CHUNK,), jnp.int32),
            pltpu.VMEM((CHUNK, D), jnp.float32),
            pltpu.VMEM_SHARED((N_PAD, D), jnp.float32),
            pltpu.SemaphoreType.DMA,
        ],
    )
    def _sc_scatter_add(h_hbm, src_hbm, dst_hbm, zeros_hbm, out_hbm,
                        sidx, didx, rows, acc, sem):
        c = lax.axis_index("c")
        s = lax.axis_index("s")
        wid = s * NC + c

        pltpu.sync_copy(zeros_hbm, rows)
        for k in range(RPT // CHUNK):
            pltpu.sync_copy(rows, acc.at[pl.ds(s * RPT + k * CHUNK, CHUNK)])
        plsc.subcore_barrier()

        def body(j, carry):
            base = wid * PER_TILE + j * CHUNK
            pltpu.sync_copy(src_hbm.at[pl.ds(base, CHUNK)], sidx)
            pltpu.sync_copy(dst_hbm.at[pl.ds(base, CHUNK)], didx)
            pltpu.async_copy(h_hbm.at[sidx], rows, sem).wait()
            pltpu.sync_copy(rows, acc.at[didx], add=True)
            return carry

        lax.fori_loop(0, NCH, body, 0)
        plsc.subcore_barrier()
        pltpu.sync_copy(acc.at[pl.ds(s * RPT, RPT)],
                        out_hbm.at[c, pl.ds(s * RPT, RPT)])

    return _sc_scatter_add


def _tc_in_body(x_ref, w_ref, b_ref, o_ref):
    o_ref[...] = jnp.maximum(
        jnp.dot(x_ref[...], w_ref[...], preferred_element_type=jnp.float32)
        + b_ref[...], 0.0)


def _tc_layer_body(p_ref, h_ref, w_ref, b_ref, g_ref, be_ref, o_ref):
    agg = p_ref[0, :N, :] + p_ref[1, :N, :] + h_ref[...]
    z = jnp.dot(agg, w_ref[...], preferred_element_type=jnp.float32) + b_ref[...]
    mu = jnp.mean(z, axis=-1, keepdims=True)
    var = jnp.mean((z - mu) ** 2, axis=-1, keepdims=True)
    zn = (z - mu) * lax.rsqrt(var + 1e-5) * g_ref[...] + be_ref[...]
    o_ref[...] = jnp.maximum(zn, 0.0)


def _tc_final_body(p_ref, h_ref, w_ref, b_ref, g_ref, be_ref,
                   batch_ref, wout_ref, bout_ref, o_ref):
    agg = p_ref[0, :N, :] + p_ref[1, :N, :] + h_ref[...]
    z = jnp.dot(agg, w_ref[...], preferred_element_type=jnp.float32) + b_ref[...]
    mu = jnp.mean(z, axis=-1, keepdims=True)
    var = jnp.mean((z - mu) ** 2, axis=-1, keepdims=True)
    zn = (z - mu) * lax.rsqrt(var + 1e-5) * g_ref[...] + be_ref[...]
    h2 = jnp.maximum(zn, 0.0)
    gids = lax.broadcasted_iota(jnp.int32, (G, N), 0)
    onehot = (batch_ref[...] == gids).astype(jnp.float32)
    pooled = jnp.dot(onehot, h2, preferred_element_type=jnp.float32)
    counts = jnp.maximum(jnp.sum(onehot, axis=1, keepdims=True), 1.0)
    mean = pooled / counts
    o_ref[...] = jnp.sum(mean * wout_ref[...], axis=1, keepdims=True) + bout_ref[...]


def kernel(node_features, edge_features, edge_index, batch,
           W_in, b_in, W1, b1, g1, be1, W2, b2, g2, be2, W_out, b_out):
    src = edge_index[0].astype(jnp.int32)
    dst = edge_index[1].astype(jnp.int32)
    pad = E_PAD - E
    ar = jnp.arange(pad, dtype=jnp.int32)
    src_p = jnp.concatenate([src, ar % N])
    dst_p = jnp.concatenate([dst, N + (ar % (N_PAD - N))])
    zeros_chunk = jnp.zeros((CHUNK, D), jnp.float32)

    b_in2 = b_in.reshape(1, D)
    b1r, g1r, be1r = b1.reshape(1, D), g1.reshape(1, D), be1.reshape(1, D)
    b2r, g2r, be2r = b2.reshape(1, D), g2.reshape(1, D), be2.reshape(1, D)
    batch_row = batch.reshape(1, N).astype(jnp.int32)
    wout_row = W_out.reshape(1, D)
    bout2 = b_out.reshape(1, 1)

    h0 = pl.pallas_call(
        _tc_in_body,
        out_shape=jax.ShapeDtypeStruct((N, D), jnp.float32),
    )(node_features, W_in, b_in2)

    sc_scatter_add = _make_sc_scatter_add()
    p1 = sc_scatter_add(h0, src_p, dst_p, zeros_chunk)

    h1 = pl.pallas_call(
        _tc_layer_body,
        out_shape=jax.ShapeDtypeStruct((N, D), jnp.float32),
    )(p1, h0, W1, b1r, g1r, be1r)

    p2 = sc_scatter_add(h1, src_p, dst_p, zeros_chunk)

    out = pl.pallas_call(
        _tc_final_body,
        out_shape=jax.ShapeDtypeStruct((G, 1), jnp.float32),
    )(p2, h1, W2, b2r, g2r, be2r, batch_row, wout_row, bout2)

    return out


# speedup vs baseline: 9.8695x; 1.6769x over previous
import functools

import jax
import jax.numpy as jnp
from jax import lax
from jax.experimental import pallas as pl
from jax.experimental.pallas import tpu as pltpu
from jax.experimental.pallas import tpu_sc as plsc

N = 10000
E = 320000
D = 128
G = 64

NC = 2
NS = 16
NW = NC * NS

CHUNK = 128
NCH = 80
CPP = NCH // 2
PER_TILE = NCH * CHUNK
E_PAD = NW * PER_TILE
N_PAD = 10112
RPT = N_PAD // NS

@functools.cache
def _make_sc_scatter_add():
    mesh = plsc.VectorSubcoreMesh(
        core_axis_name="c", subcore_axis_name="s", num_cores=NC, num_subcores=NS
    )

    @functools.partial(
        pl.kernel,
        out_type=jax.ShapeDtypeStruct((NC, N_PAD, D), jnp.float32),
        mesh=mesh,
        scratch_types=[
            pltpu.VMEM((CPP, 2, CHUNK), jnp.int32),
            pltpu.VMEM((CHUNK, D), jnp.float32),
            pltpu.VMEM((CHUNK, D), jnp.float32),
            pltpu.VMEM_SHARED((N_PAD, D), jnp.float32),
            pltpu.SemaphoreType.DMA,
            pltpu.SemaphoreType.DMA,
            pltpu.SemaphoreType.DMA,
            pltpu.SemaphoreType.DMA,
        ],
    )
    def _sc_scatter_add(h_hbm, idx_hbm, zeros_hbm, out_hbm,
                        idx2, rows_a, rows_b, acc,
                        gsem_a, gsem_b, ssem_a, ssem_b):
        c = lax.axis_index("c")
        s = lax.axis_index("s")
        wid = s * NC + c

        pltpu.sync_copy(zeros_hbm, rows_a)
        off = 0
        while off < RPT:
            step = min(CHUNK, RPT - off)
            pltpu.sync_copy(rows_a.at[pl.ds(0, step)],
                            acc.at[pl.ds(s * RPT + off, step)])
            off += step
        plsc.subcore_barrier()

        nt = CPP // 2
        for ph in range(NCH // CPP):
            pltpu.sync_copy(idx_hbm.at[pl.ds(wid * NCH + ph * CPP, CPP)], idx2)
            pltpu.async_copy(h_hbm.at[idx2.at[0, 0]], rows_a, gsem_a)

            def body(t, carry):
                j0 = 2 * t
                pltpu.make_async_copy(h_hbm.at[idx2.at[j0, 0]], rows_a,
                                      gsem_a).wait()
                pltpu.async_copy(rows_a, acc.at[idx2.at[j0, 1]], ssem_a,
                                 add=True)

                @pl.when(t > 0)
                def _():
                    pltpu.make_async_copy(rows_b, acc.at[idx2.at[j0 - 1, 1]],
                                          ssem_b).wait()
                pltpu.async_copy(h_hbm.at[idx2.at[j0 + 1, 0]], rows_b, gsem_b)

                pltpu.make_async_copy(h_hbm.at[idx2.at[j0 + 1, 0]], rows_b,
                                      gsem_b).wait()
                pltpu.async_copy(rows_b, acc.at[idx2.at[j0 + 1, 1]], ssem_b,
                                 add=True)

                pltpu.make_async_copy(rows_a, acc.at[idx2.at[j0, 1]],
                                      ssem_a).wait()

                @pl.when(t < nt - 1)
                def _():
                    pltpu.async_copy(h_hbm.at[idx2.at[j0 + 2, 0]], rows_a,
                                     gsem_a)
                return carry

            lax.fori_loop(0, nt, body, 0)
            pltpu.make_async_copy(rows_b, acc.at[idx2.at[CPP - 1, 1]],
                                  ssem_b).wait()
        plsc.subcore_barrier()
        pltpu.sync_copy(acc.at[pl.ds(s * RPT, RPT)],
                        out_hbm.at[c, pl.ds(s * RPT, RPT)])

    return _sc_scatter_add


def _tc_in_body(x_ref, w_ref, b_ref, o_ref):
    o_ref[...] = jnp.maximum(
        jnp.dot(x_ref[...], w_ref[...], preferred_element_type=jnp.float32)
        + b_ref[...], 0.0)


def _tc_layer_body(p_ref, h_ref, w_ref, b_ref, g_ref, be_ref, o_ref):
    agg = p_ref[0, :N, :] + p_ref[1, :N, :] + h_ref[...]
    z = jnp.dot(agg, w_ref[...], preferred_element_type=jnp.float32) + b_ref[...]
    mu = jnp.mean(z, axis=-1, keepdims=True)
    var = jnp.mean((z - mu) ** 2, axis=-1, keepdims=True)
    zn = (z - mu) * lax.rsqrt(var + 1e-5) * g_ref[...] + be_ref[...]
    o_ref[...] = jnp.maximum(zn, 0.0)


def _tc_final_body(p_ref, h_ref, w_ref, b_ref, g_ref, be_ref,
                   batch_ref, wout_ref, bout_ref, o_ref):
    agg = p_ref[0, :N, :] + p_ref[1, :N, :] + h_ref[...]
    z = jnp.dot(agg, w_ref[...], preferred_element_type=jnp.float32) + b_ref[...]
    mu = jnp.mean(z, axis=-1, keepdims=True)
    var = jnp.mean((z - mu) ** 2, axis=-1, keepdims=True)
    zn = (z - mu) * lax.rsqrt(var + 1e-5) * g_ref[...] + be_ref[...]
    h2 = jnp.maximum(zn, 0.0)
    gids = lax.broadcasted_iota(jnp.int32, (G, N), 0)
    onehot = (batch_ref[...] == gids).astype(jnp.float32)
    pooled = jnp.dot(onehot, h2, preferred_element_type=jnp.float32)
    counts = jnp.maximum(jnp.sum(onehot, axis=1, keepdims=True), 1.0)
    mean = pooled / counts
    o_ref[...] = jnp.sum(mean * wout_ref[...], axis=1, keepdims=True) + bout_ref[...]


def kernel(node_features, edge_features, edge_index, batch,
           W_in, b_in, W1, b1, g1, be1, W2, b2, g2, be2, W_out, b_out):
    src = edge_index[0].astype(jnp.int32)
    dst = edge_index[1].astype(jnp.int32)
    pad = E_PAD - E
    ar = jnp.arange(pad, dtype=jnp.int32)
    src_p = jnp.concatenate([src, ar % N]).reshape(NW * NCH, 1, CHUNK)
    dst_p = jnp.concatenate([dst, N + (ar % (N_PAD - N))]).reshape(NW * NCH, 1, CHUNK)
    idx_p = jnp.concatenate([src_p, dst_p], axis=1)
    zeros_chunk = jnp.zeros((CHUNK, D), jnp.float32)

    b_in2 = b_in.reshape(1, D)
    b1r, g1r, be1r = b1.reshape(1, D), g1.reshape(1, D), be1.reshape(1, D)
    b2r, g2r, be2r = b2.reshape(1, D), g2.reshape(1, D), be2.reshape(1, D)
    batch_row = batch.reshape(1, N).astype(jnp.int32)
    wout_row = W_out.reshape(1, D)
    bout2 = b_out.reshape(1, 1)

    h0 = pl.pallas_call(
        _tc_in_body,
        out_shape=jax.ShapeDtypeStruct((N, D), jnp.float32),
    )(node_features, W_in, b_in2)

    sc_scatter_add = _make_sc_scatter_add()
    p1 = sc_scatter_add(h0, idx_p, zeros_chunk)

    h1 = pl.pallas_call(
        _tc_layer_body,
        out_shape=jax.ShapeDtypeStruct((N, D), jnp.float32),
    )(p1, h0, W1, b1r, g1r, be1r)

    p2 = sc_scatter_add(h1, idx_p, zeros_chunk)

    out = pl.pallas_call(
        _tc_final_body,
        out_shape=jax.ShapeDtypeStruct((G, 1), jnp.float32),
    )(p2, h1, W2, b2r, g2r, be2r, batch_row, wout_row, bout2)

    return out


# speedup vs baseline: 10.0798x; 1.0213x over previous
import functools

import jax
import jax.numpy as jnp
from jax import lax
from jax.experimental import pallas as pl
from jax.experimental.pallas import tpu as pltpu
from jax.experimental.pallas import tpu_sc as plsc

N = 10000
E = 320000
D = 128
G = 64

NC = 2
NS = 16
NW = NC * NS

CHUNK = 128
NCH = 80
CPP = NCH // 2
PER_TILE = NCH * CHUNK
E_PAD = NW * PER_TILE
N_PAD = 10112
RPT = N_PAD // NS

@functools.cache
def _make_sc_scatter_add():
    mesh = plsc.VectorSubcoreMesh(
        core_axis_name="c", subcore_axis_name="s", num_cores=NC, num_subcores=NS
    )

    @functools.partial(
        pl.kernel,
        out_type=jax.ShapeDtypeStruct((NC, N_PAD, D), jnp.float32),
        mesh=mesh,
        scratch_types=[
            pltpu.VMEM((CPP, 2, CHUNK), jnp.int32),
            pltpu.VMEM((CHUNK, D), jnp.float32),
            pltpu.VMEM((CHUNK, D), jnp.float32),
            pltpu.VMEM_SHARED((N_PAD, D), jnp.float32),
            pltpu.SemaphoreType.DMA,
            pltpu.SemaphoreType.DMA,
            pltpu.SemaphoreType.DMA,
            pltpu.SemaphoreType.DMA,
        ],
    )
    def _sc_scatter_add(h_hbm, idx_hbm, out_hbm,
                        idx2, rows_a, rows_b, acc,
                        gsem_a, gsem_b, ssem_a, ssem_b):
        c = lax.axis_index("c")
        s = lax.axis_index("s")
        wid = s * NC + c

        zz = jnp.zeros((16,), jnp.float32)

        def zbody(i, carry):
            for k in range(D // 16):
                rows_a[i, pl.ds(k * 16, 16)] = zz
            return carry

        lax.fori_loop(0, CHUNK, zbody, 0)
        steps = []
        off = 0
        while off < RPT:
            step = min(CHUNK, RPT - off)
            steps.append((off, step))
            off += step
        for off, step in steps:
            pltpu.async_copy(rows_a.at[pl.ds(0, step)],
                             acc.at[pl.ds(s * RPT + off, step)], ssem_a)
        pltpu.sync_copy(idx_hbm.at[pl.ds(wid * NCH, CPP)], idx2)
        for off, step in steps:
            pltpu.make_async_copy(rows_a.at[pl.ds(0, step)],
                                  acc.at[pl.ds(s * RPT + off, step)],
                                  ssem_a).wait()
        plsc.subcore_barrier()

        nt = CPP // 2
        for ph in range(NCH // CPP):
            if ph > 0:
                pltpu.sync_copy(
                    idx_hbm.at[pl.ds(wid * NCH + ph * CPP, CPP)], idx2)
            pltpu.async_copy(h_hbm.at[idx2.at[0, 0]], rows_a, gsem_a)

            def body(t, carry):
                j0 = 2 * t
                pltpu.make_async_copy(h_hbm.at[idx2.at[j0, 0]], rows_a,
                                      gsem_a).wait()
                pltpu.async_copy(rows_a, acc.at[idx2.at[j0, 1]], ssem_a,
                                 add=True)

                @pl.when(t > 0)
                def _():
                    pltpu.make_async_copy(rows_b, acc.at[idx2.at[j0 - 1, 1]],
                                          ssem_b).wait()
                pltpu.async_copy(h_hbm.at[idx2.at[j0 + 1, 0]], rows_b, gsem_b)

                pltpu.make_async_copy(h_hbm.at[idx2.at[j0 + 1, 0]], rows_b,
                                      gsem_b).wait()
                pltpu.async_copy(rows_b, acc.at[idx2.at[j0 + 1, 1]], ssem_b,
                                 add=True)

                pltpu.make_async_copy(rows_a, acc.at[idx2.at[j0, 1]],
                                      ssem_a).wait()

                @pl.when(t < nt - 1)
                def _():
                    pltpu.async_copy(h_hbm.at[idx2.at[j0 + 2, 0]], rows_a,
                                     gsem_a)
                return carry

            lax.fori_loop(0, nt, body, 0)
            pltpu.make_async_copy(rows_b, acc.at[idx2.at[CPP - 1, 1]],
                                  ssem_b).wait()
        plsc.subcore_barrier()
        pltpu.sync_copy(acc.at[pl.ds(s * RPT, RPT)],
                        out_hbm.at[c, pl.ds(s * RPT, RPT)])

    return _sc_scatter_add


def _tc_in_body(x_ref, w_ref, b_ref, o_ref):
    o_ref[...] = jnp.maximum(
        jnp.dot(x_ref[...], w_ref[...], preferred_element_type=jnp.float32)
        + b_ref[...], 0.0)


def _tc_layer_body(p_ref, h_ref, w_ref, b_ref, g_ref, be_ref, o_ref):
    agg = p_ref[0] + p_ref[1] + h_ref[...]
    z = jnp.dot(agg, w_ref[...], preferred_element_type=jnp.float32) + b_ref[...]
    mu = jnp.mean(z, axis=-1, keepdims=True)
    var = jnp.mean((z - mu) ** 2, axis=-1, keepdims=True)
    zn = (z - mu) * lax.rsqrt(var + 1e-5) * g_ref[...] + be_ref[...]
    o_ref[...] = jnp.maximum(zn, 0.0)


def _tc_pool_body(h_ref, batch_ref, wout_ref, bout_ref, o_ref):
    gids = lax.broadcasted_iota(jnp.int32, (G, N), 0)
    onehot = (batch_ref[...] == gids).astype(jnp.float32)
    pooled = jnp.dot(onehot, h_ref[...], preferred_element_type=jnp.float32)
    counts = jnp.maximum(jnp.sum(onehot, axis=1, keepdims=True), 1.0)
    mean = pooled / counts
    o_ref[...] = jnp.sum(mean * wout_ref[...], axis=1, keepdims=True) + bout_ref[...]


BN = 2000


def _tc_in(x, w, b2):
    return pl.pallas_call(
        _tc_in_body,
        grid=(N // BN,),
        in_specs=[
            pl.BlockSpec((BN, D), lambda i: (i, 0)),
            pl.BlockSpec((D, D), lambda i: (0, 0)),
            pl.BlockSpec((1, D), lambda i: (0, 0)),
        ],
        out_specs=pl.BlockSpec((BN, D), lambda i: (i, 0)),
        out_shape=jax.ShapeDtypeStruct((N, D), jnp.float32),
    )(x, w, b2)


def _tc_layer(p, h, w, b2, g2, be2):
    return pl.pallas_call(
        _tc_layer_body,
        grid=(N // BN,),
        in_specs=[
            pl.BlockSpec((NC, BN, D), lambda i: (0, i, 0)),
            pl.BlockSpec((BN, D), lambda i: (i, 0)),
            pl.BlockSpec((D, D), lambda i: (0, 0)),
            pl.BlockSpec((1, D), lambda i: (0, 0)),
            pl.BlockSpec((1, D), lambda i: (0, 0)),
            pl.BlockSpec((1, D), lambda i: (0, 0)),
        ],
        out_specs=pl.BlockSpec((BN, D), lambda i: (i, 0)),
        out_shape=jax.ShapeDtypeStruct((N, D), jnp.float32),
    )(p, h, w, b2, g2, be2)


def kernel(node_features, edge_features, edge_index, batch,
           W_in, b_in, W1, b1, g1, be1, W2, b2, g2, be2, W_out, b_out):
    src = edge_index[0].astype(jnp.int32)
    dst = edge_index[1].astype(jnp.int32)
    pad = E_PAD - E
    ar = jnp.arange(pad, dtype=jnp.int32)
    src_p = jnp.concatenate([src, ar % N]).reshape(NW * NCH, 1, CHUNK)
    dst_p = jnp.concatenate([dst, N + (ar % (N_PAD - N))]).reshape(NW * NCH, 1, CHUNK)
    idx_p = jnp.concatenate([src_p, dst_p], axis=1)

    b_in2 = b_in.reshape(1, D)
    b1r, g1r, be1r = b1.reshape(1, D), g1.reshape(1, D), be1.reshape(1, D)
    b2r, g2r, be2r = b2.reshape(1, D), g2.reshape(1, D), be2.reshape(1, D)
    batch_row = batch.reshape(1, N).astype(jnp.int32)
    wout_row = W_out.reshape(1, D)
    bout2 = b_out.reshape(1, 1)

    h0 = _tc_in(node_features, W_in, b_in2)

    sc_scatter_add = _make_sc_scatter_add()
    p1 = sc_scatter_add(h0, idx_p)

    h1 = _tc_layer(p1, h0, W1, b1r, g1r, be1r)

    p2 = sc_scatter_add(h1, idx_p)

    h2 = _tc_layer(p2, h1, W2, b2r, g2r, be2r)

    out = pl.pallas_call(
        _tc_pool_body,
        out_shape=jax.ShapeDtypeStruct((G, 1), jnp.float32),
    )(h2, batch_row, wout_row, bout2)

    return out


# speedup vs baseline: 12.2573x; 1.2160x over previous
import functools

import jax
import jax.numpy as jnp
from jax import lax
from jax.experimental import pallas as pl
from jax.experimental.pallas import tpu as pltpu
from jax.experimental.pallas import tpu_sc as plsc

N = 10000
E = 320000
D = 128
G = 64

NC = 2
NS = 16
NW = NC * NS

CHUNK = 120
NCH = 84
PER_TILE = NCH * CHUNK
E_PAD = NW * PER_TILE
N_PAD = 10112
RPT = N_PAD // NS
NRB = 3
NIB = 6
UNROLL = 6

@functools.cache
def _make_sc_scatter_add():
    mesh = plsc.VectorSubcoreMesh(
        core_axis_name="c", subcore_axis_name="s", num_cores=NC, num_subcores=NS
    )

    @functools.partial(
        pl.kernel,
        out_type=jax.ShapeDtypeStruct((NC, N_PAD, D), jnp.float32),
        mesh=mesh,
        scratch_types=[
            [pltpu.VMEM((CHUNK, D), jnp.float32)] * NRB,
            [pltpu.VMEM((1, 1, CHUNK), jnp.int32)] * NIB,
            [pltpu.VMEM((1, 1, CHUNK), jnp.int32)] * NIB,
            pltpu.VMEM_SHARED((N_PAD, D), jnp.float32),
            [pltpu.SemaphoreType.DMA] * NRB,
            [pltpu.SemaphoreType.DMA] * NRB,
            [pltpu.SemaphoreType.DMA] * NIB,
        ],
    )
    def _sc_scatter_add(h_hbm, src_hbm, dst_hbm, out_hbm,
                        rows, sbuf, dbuf, acc, gsem, ssem, isem):
        c = lax.axis_index("c")
        s = lax.axis_index("s")
        wid = s * NC + c

        def idx_load(j, y):
            row = wid * NCH + j
            pltpu.async_copy(src_hbm.at[pl.ds(row, 1)], sbuf[y], isem[y])
            pltpu.async_copy(dst_hbm.at[pl.ds(row, 1)], dbuf[y], isem[y])

        def idx_wait(y):
            pltpu.make_async_copy(src_hbm.at[pl.ds(0, 1)], sbuf[y],
                                  isem[y]).wait()
            pltpu.make_async_copy(src_hbm.at[pl.ds(0, 1)], dbuf[y],
                                  isem[y]).wait()

        def gather_start(x, y):
            pltpu.async_copy(h_hbm.at[sbuf[y].at[0, 0]], rows[x], gsem[x])

        def gather_wait(x, y):
            pltpu.make_async_copy(h_hbm.at[sbuf[y].at[0, 0]], rows[x],
                                  gsem[x]).wait()

        def scatter_start(x, y):
            pltpu.async_copy(rows[x], acc.at[dbuf[y].at[0, 0]], ssem[x],
                             add=True)

        def scatter_wait(x, y):
            pltpu.make_async_copy(rows[x], acc.at[dbuf[y].at[0, 0]],
                                  ssem[x]).wait()

        zz = jnp.zeros((16,), jnp.float32)

        def zbody(i, carry):
            for k in range(D // 16):
                rows[0][i, pl.ds(k * 16, 16)] = zz
            return carry

        lax.fori_loop(0, CHUNK, zbody, 0)
        steps = []
        off = 0
        while off < RPT:
            step = min(CHUNK, RPT - off)
            steps.append((off, step))
            off += step
        for off, step in steps:
            pltpu.async_copy(rows[0].at[pl.ds(0, step)],
                             acc.at[pl.ds(s * RPT + off, step)], ssem[0])
        for dj in range(NRB):
            idx_load(dj, dj % NIB)
        for off, step in steps:
            pltpu.make_async_copy(rows[0].at[pl.ds(0, step)],
                                  acc.at[pl.ds(s * RPT + off, step)],
                                  ssem[0]).wait()
        plsc.subcore_barrier()

        nt = NCH // UNROLL

        def body(t, carry):
            j0 = t * UNROLL
            for dj in range(UNROLL):
                x = dj % NRB
                y = dj % NIB
                j = j0 + dj

                if dj >= NRB:
                    scatter_wait(x, (dj - NRB) % NIB)
                else:
                    @pl.when(t > 0)
                    def _():
                        scatter_wait(x, (dj - NRB) % NIB)
                yn = (dj + NRB) % NIB
                if dj + NRB < UNROLL:
                    idx_load(j + NRB, yn)
                else:
                    @pl.when(t < nt - 1)
                    def _():
                        idx_load(j + NRB, yn)

                idx_wait(y)
                gather_start(x, y)

                xp = (dj - 1) % NRB
                yp = (dj - 1) % NIB
                if dj == 0:
                    @pl.when(t > 0)
                    def _():
                        gather_wait(xp, yp)
                        scatter_start(xp, yp)
                else:
                    gather_wait(xp, yp)
                    scatter_start(xp, yp)
            return carry

        lax.fori_loop(0, nt, body, 0)
        xl = (NCH - 1) % NRB
        yl = (NCH - 1) % NIB
        gather_wait(xl, yl)
        scatter_start(xl, yl)
        for k in range(NRB):
            j = NCH - NRB + k
            scatter_wait(j % NRB, j % NIB)
        plsc.subcore_barrier()
        pltpu.sync_copy(acc.at[pl.ds(s * RPT, RPT)],
                        out_hbm.at[c, pl.ds(s * RPT, RPT)])

    return _sc_scatter_add


def _tc_in_body(x_ref, w_ref, b_ref, o_ref):
    o_ref[...] = jnp.maximum(
        jnp.dot(x_ref[...], w_ref[...], preferred_element_type=jnp.float32)
        + b_ref[...], 0.0)


def _tc_layer_body(p_ref, h_ref, w_ref, b_ref, g_ref, be_ref, o_ref):
    agg = p_ref[0] + p_ref[1] + h_ref[...]
    z = jnp.dot(agg, w_ref[...], preferred_element_type=jnp.float32) + b_ref[...]
    mu = jnp.mean(z, axis=-1, keepdims=True)
    var = jnp.mean((z - mu) ** 2, axis=-1, keepdims=True)
    zn = (z - mu) * lax.rsqrt(var + 1e-5) * g_ref[...] + be_ref[...]
    o_ref[...] = jnp.maximum(zn, 0.0)


def _tc_pool_body(h_ref, batch_ref, wout_ref, bout_ref, o_ref):
    gids = lax.broadcasted_iota(jnp.int32, (G, N), 0)
    onehot = (batch_ref[...] == gids).astype(jnp.float32)
    pooled = jnp.dot(onehot, h_ref[...], preferred_element_type=jnp.float32)
    counts = jnp.maximum(jnp.sum(onehot, axis=1, keepdims=True), 1.0)
    mean = pooled / counts
    o_ref[...] = jnp.sum(mean * wout_ref[...], axis=1, keepdims=True) + bout_ref[...]


BN = 2000


def _tc_in(x, w, b2):
    return pl.pallas_call(
        _tc_in_body,
        grid=(N // BN,),
        in_specs=[
            pl.BlockSpec((BN, D), lambda i: (i, 0)),
            pl.BlockSpec((D, D), lambda i: (0, 0)),
            pl.BlockSpec((1, D), lambda i: (0, 0)),
        ],
        out_specs=pl.BlockSpec((BN, D), lambda i: (i, 0)),
        out_shape=jax.ShapeDtypeStruct((N, D), jnp.float32),
    )(x, w, b2)


def _tc_layer(p, h, w, b2, g2, be2):
    return pl.pallas_call(
        _tc_layer_body,
        grid=(N // BN,),
        in_specs=[
            pl.BlockSpec((NC, BN, D), lambda i: (0, i, 0)),
            pl.BlockSpec((BN, D), lambda i: (i, 0)),
            pl.BlockSpec((D, D), lambda i: (0, 0)),
            pl.BlockSpec((1, D), lambda i: (0, 0)),
            pl.BlockSpec((1, D), lambda i: (0, 0)),
            pl.BlockSpec((1, D), lambda i: (0, 0)),
        ],
        out_specs=pl.BlockSpec((BN, D), lambda i: (i, 0)),
        out_shape=jax.ShapeDtypeStruct((N, D), jnp.float32),
    )(p, h, w, b2, g2, be2)


def kernel(node_features, edge_features, edge_index, batch,
           W_in, b_in, W1, b1, g1, be1, W2, b2, g2, be2, W_out, b_out):
    src = edge_index[0].astype(jnp.int32)
    dst = edge_index[1].astype(jnp.int32)
    pad = E_PAD - E
    ar = jnp.arange(pad, dtype=jnp.int32)
    src_p = jnp.concatenate([src, ar % N]).reshape(NW * NCH, 1, CHUNK)
    dst_p = jnp.concatenate([dst, N + (ar % (N_PAD - N))]).reshape(NW * NCH, 1, CHUNK)

    b_in2 = b_in.reshape(1, D)
    b1r, g1r, be1r = b1.reshape(1, D), g1.reshape(1, D), be1.reshape(1, D)
    b2r, g2r, be2r = b2.reshape(1, D), g2.reshape(1, D), be2.reshape(1, D)
    batch_row = batch.reshape(1, N).astype(jnp.int32)
    wout_row = W_out.reshape(1, D)
    bout2 = b_out.reshape(1, 1)

    h0 = _tc_in(node_features, W_in, b_in2)

    sc_scatter_add = _make_sc_scatter_add()
    p1 = sc_scatter_add(h0, src_p, dst_p)

    h1 = _tc_layer(p1, h0, W1, b1r, g1r, be1r)

    p2 = sc_scatter_add(h1, src_p, dst_p)

    h2 = _tc_layer(p2, h1, W2, b2r, g2r, be2r)

    out = pl.pallas_call(
        _tc_pool_body,
        out_shape=jax.ShapeDtypeStruct((G, 1), jnp.float32),
    )(h2, batch_row, wout_row, bout2)

    return out


# speedup vs baseline: 12.4405x; 1.0149x over previous
import functools

import jax
import jax.numpy as jnp
from jax import lax
from jax.experimental import pallas as pl
from jax.experimental.pallas import tpu as pltpu
from jax.experimental.pallas import tpu_sc as plsc

N = 10000
E = 320000
D = 128
G = 64

NC = 2
NS = 16
NW = NC * NS

CHUNK = 120
NCH = 84
PER_TILE = NCH * CHUNK
E_PAD = NW * PER_TILE
N_PAD = 10112
RPT = N_PAD // NS
NRB = 3
NIB = 6
UNROLL = 6

@functools.cache
def _make_sc_scatter_add():
    mesh = plsc.VectorSubcoreMesh(
        core_axis_name="c", subcore_axis_name="s", num_cores=NC, num_subcores=NS
    )

    @functools.partial(
        pl.kernel,
        out_type=jax.ShapeDtypeStruct((NC, N_PAD, D), jnp.float32),
        mesh=mesh,
        scratch_types=[
            [pltpu.VMEM((CHUNK, D), jnp.float32)] * NRB,
            [pltpu.VMEM((1, 1, CHUNK), jnp.int32)] * NIB,
            [pltpu.VMEM((1, 1, CHUNK), jnp.int32)] * NIB,
            pltpu.VMEM_SHARED((N_PAD, D), jnp.float32),
            [pltpu.SemaphoreType.DMA] * NRB,
            [pltpu.SemaphoreType.DMA] * NRB,
            [pltpu.SemaphoreType.DMA] * NIB,
        ],
    )
    def _sc_scatter_add(h_hbm, src_hbm, dst_hbm, out_hbm,
                        rows, sbuf, dbuf, acc, gsem, ssem, isem):
        c = lax.axis_index("c")
        s = lax.axis_index("s")
        wid = s * NC + c

        def idx_load(j, y):
            row = wid * NCH + j
            pltpu.async_copy(src_hbm.at[pl.ds(row, 1)], sbuf[y], isem[y])
            pltpu.async_copy(dst_hbm.at[pl.ds(row, 1)], dbuf[y], isem[y])

        def idx_wait(y):
            pltpu.make_async_copy(src_hbm.at[pl.ds(0, 1)], sbuf[y],
                                  isem[y]).wait()
            pltpu.make_async_copy(src_hbm.at[pl.ds(0, 1)], dbuf[y],
                                  isem[y]).wait()

        def gather_start(x, y):
            pltpu.async_copy(h_hbm.at[sbuf[y].at[0, 0]], rows[x], gsem[x])

        def gather_wait(x, y):
            pltpu.make_async_copy(h_hbm.at[sbuf[y].at[0, 0]], rows[x],
                                  gsem[x]).wait()

        def scatter_start(x, y):
            pltpu.async_copy(rows[x], acc.at[dbuf[y].at[0, 0]], ssem[x],
                             add=True)

        def scatter_wait(x, y):
            pltpu.make_async_copy(rows[x], acc.at[dbuf[y].at[0, 0]],
                                  ssem[x]).wait()

        zz = jnp.zeros((16,), jnp.float32)

        def zbody(i, carry):
            for k in range(D // 16):
                rows[0][i, pl.ds(k * 16, 16)] = zz
            return carry

        lax.fori_loop(0, CHUNK, zbody, 0)
        steps = []
        off = 0
        while off < RPT:
            step = min(CHUNK, RPT - off)
            steps.append((off, step))
            off += step
        for off, step in steps:
            pltpu.async_copy(rows[0].at[pl.ds(0, step)],
                             acc.at[pl.ds(s * RPT + off, step)], ssem[0])
        for dj in range(NRB):
            idx_load(dj, dj % NIB)
        for off, step in steps:
            pltpu.make_async_copy(rows[0].at[pl.ds(0, step)],
                                  acc.at[pl.ds(s * RPT + off, step)],
                                  ssem[0]).wait()
        plsc.subcore_barrier()

        nt = NCH // UNROLL

        def body(t, carry):
            j0 = t * UNROLL
            for dj in range(UNROLL):
                x = dj % NRB
                y = dj % NIB
                j = j0 + dj

                if dj >= NRB:
                    scatter_wait(x, (dj - NRB) % NIB)
                else:
                    @pl.when(t > 0)
                    def _():
                        scatter_wait(x, (dj - NRB) % NIB)
                yn = (dj + NRB) % NIB
                if dj + NRB < UNROLL:
                    idx_load(j + NRB, yn)
                else:
                    @pl.when(t < nt - 1)
                    def _():
                        idx_load(j + NRB, yn)

                idx_wait(y)
                gather_start(x, y)

                xp = (dj - 1) % NRB
                yp = (dj - 1) % NIB
                if dj == 0:
                    @pl.when(t > 0)
                    def _():
                        gather_wait(xp, yp)
                        scatter_start(xp, yp)
                else:
                    gather_wait(xp, yp)
                    scatter_start(xp, yp)
            return carry

        lax.fori_loop(0, nt, body, 0)
        xl = (NCH - 1) % NRB
        yl = (NCH - 1) % NIB
        gather_wait(xl, yl)
        scatter_start(xl, yl)
        for k in range(NRB):
            j = NCH - NRB + k
            scatter_wait(j % NRB, j % NIB)
        plsc.subcore_barrier()
        pltpu.sync_copy(acc.at[pl.ds(s * RPT, RPT)],
                        out_hbm.at[c, pl.ds(s * RPT, RPT)])

    return _sc_scatter_add


def _tc_in_body(x_ref, w_ref, b_ref, o_ref):
    o_ref[...] = jnp.maximum(
        jnp.dot(x_ref[...], w_ref[...], preferred_element_type=jnp.float32)
        + b_ref[...], 0.0)


def _tc_layer_body(p_ref, h_ref, w_ref, b_ref, g_ref, be_ref, o_ref):
    agg = p_ref[0] + p_ref[1] + h_ref[...]
    z = jnp.dot(agg, w_ref[...], preferred_element_type=jnp.float32) + b_ref[...]
    mu = jnp.mean(z, axis=-1, keepdims=True)
    var = jnp.mean((z - mu) ** 2, axis=-1, keepdims=True)
    zn = (z - mu) * lax.rsqrt(var + 1e-5) * g_ref[...] + be_ref[...]
    o_ref[...] = jnp.maximum(zn, 0.0)


def _tc_layer_pool_body(p_ref, h_ref, w_ref, b_ref, g_ref, be_ref,
                        batch_ref, wout_ref, bout_ref, o_ref,
                        pooled_ref, cnt_ref):
    i = pl.program_id(0)
    agg = p_ref[0] + p_ref[1] + h_ref[...]
    z = jnp.dot(agg, w_ref[...], preferred_element_type=jnp.float32) + b_ref[...]
    mu = jnp.mean(z, axis=-1, keepdims=True)
    var = jnp.mean((z - mu) ** 2, axis=-1, keepdims=True)
    zn = (z - mu) * lax.rsqrt(var + 1e-5) * g_ref[...] + be_ref[...]
    h2 = jnp.maximum(zn, 0.0)
    gids = lax.broadcasted_iota(jnp.int32, (G, BN), 0)
    onehot = (batch_ref[0] == gids).astype(jnp.float32)
    pooled = jnp.dot(onehot, h2, preferred_element_type=jnp.float32)
    counts = jnp.sum(onehot, axis=1, keepdims=True)

    @pl.when(i == 0)
    def _():
        pooled_ref[...] = pooled
        cnt_ref[...] = counts

    @pl.when(i > 0)
    def _():
        pooled_ref[...] += pooled
        cnt_ref[...] += counts

    @pl.when(i == pl.num_programs(0) - 1)
    def _():
        mean = pooled_ref[...] / jnp.maximum(cnt_ref[...], 1.0)
        o_ref[...] = (jnp.sum(mean * wout_ref[...], axis=1, keepdims=True)
                      + bout_ref[...])


BN = 2000


def _tc_in(x, w, b2):
    return pl.pallas_call(
        _tc_in_body,
        grid=(N // BN,),
        in_specs=[
            pl.BlockSpec((BN, D), lambda i: (i, 0)),
            pl.BlockSpec((D, D), lambda i: (0, 0)),
            pl.BlockSpec((1, D), lambda i: (0, 0)),
        ],
        out_specs=pl.BlockSpec((BN, D), lambda i: (i, 0)),
        out_shape=jax.ShapeDtypeStruct((N, D), jnp.float32),
    )(x, w, b2)


def _tc_layer(p, h, w, b2, g2, be2):
    return pl.pallas_call(
        _tc_layer_body,
        grid=(N // BN,),
        in_specs=[
            pl.BlockSpec((NC, BN, D), lambda i: (0, i, 0)),
            pl.BlockSpec((BN, D), lambda i: (i, 0)),
            pl.BlockSpec((D, D), lambda i: (0, 0)),
            pl.BlockSpec((1, D), lambda i: (0, 0)),
            pl.BlockSpec((1, D), lambda i: (0, 0)),
            pl.BlockSpec((1, D), lambda i: (0, 0)),
        ],
        out_specs=pl.BlockSpec((BN, D), lambda i: (i, 0)),
        out_shape=jax.ShapeDtypeStruct((N, D), jnp.float32),
    )(p, h, w, b2, g2, be2)


def kernel(node_features, edge_features, edge_index, batch,
           W_in, b_in, W1, b1, g1, be1, W2, b2, g2, be2, W_out, b_out):
    src = edge_index[0].astype(jnp.int32)
    dst = edge_index[1].astype(jnp.int32)
    pad = E_PAD - E
    ar = jnp.arange(pad, dtype=jnp.int32)
    src_p = jnp.concatenate([src, ar % N]).reshape(NW * NCH, 1, CHUNK)
    dst_p = jnp.concatenate([dst, N + (ar % (N_PAD - N))]).reshape(NW * NCH, 1, CHUNK)

    b_in2 = b_in.reshape(1, D)
    b1r, g1r, be1r = b1.reshape(1, D), g1.reshape(1, D), be1.reshape(1, D)
    b2r, g2r, be2r = b2.reshape(1, D), g2.reshape(1, D), be2.reshape(1, D)
    batch_blk = batch.reshape(N // BN, 1, BN).astype(jnp.int32)
    wout_row = W_out.reshape(1, D)
    bout2 = b_out.reshape(1, 1)

    h0 = _tc_in(node_features, W_in, b_in2)

    sc_scatter_add = _make_sc_scatter_add()
    p1 = sc_scatter_add(h0, src_p, dst_p)

    h1 = _tc_layer(p1, h0, W1, b1r, g1r, be1r)

    p2 = sc_scatter_add(h1, src_p, dst_p)

    out = pl.pallas_call(
        _tc_layer_pool_body,
        grid=(N // BN,),
        in_specs=[
            pl.BlockSpec((NC, BN, D), lambda i: (0, i, 0)),
            pl.BlockSpec((BN, D), lambda i: (i, 0)),
            pl.BlockSpec((D, D), lambda i: (0, 0)),
            pl.BlockSpec((1, D), lambda i: (0, 0)),
            pl.BlockSpec((1, D), lambda i: (0, 0)),
            pl.BlockSpec((1, D), lambda i: (0, 0)),
            pl.BlockSpec((1, 1, BN), lambda i: (i, 0, 0)),
            pl.BlockSpec((1, D), lambda i: (0, 0)),
            pl.BlockSpec((1, 1), lambda i: (0, 0)),
        ],
        out_specs=pl.BlockSpec((G, 1), lambda i: (0, 0)),
        out_shape=jax.ShapeDtypeStruct((G, 1), jnp.float32),
        scratch_shapes=[
            pltpu.VMEM((G, D), jnp.float32),
            pltpu.VMEM((G, 1), jnp.float32),
        ],
    )(p2, h1, W2, b2r, g2r, be2r, batch_blk, wout_row, bout2)

    return out
